# baseline (speedup 1.0000x reference)
import functools

import jax
import jax.numpy as jnp
import numpy as np
from jax import lax
from jax.experimental import pallas as pl
from jax.experimental.pallas import tpu as pltpu
from jax.experimental.pallas import tpu_sc as plsc

_PATCH_L = 16
_PATCH_C = 8
_NUM_PATCHES = 256
_OUT_DIM = 128
_PATCH_FLAT = _PATCH_L * _PATCH_C

_NC, _NS = 2, 16
_NW = _NC * _NS
_CHUNK = 16
_CROWS = _CHUNK * 8
_NBUF = 4


def _sc_gather_patches(xp, sl, sc, L, C):
    W = xp.shape[0]
    NP = sl.shape[0]
    P = NP // _NW
    n_chunks = P // _CHUNK
    lb_max = L // 128 - 1

    mesh = plsc.VectorSubcoreMesh(core_axis_name="c", subcore_axis_name="s")

    @functools.partial(
        pl.kernel,
        out_type=jax.ShapeDtypeStruct((NP * _PATCH_FLAT,), jnp.float32),
        mesh=mesh,
        scratch_types=[
            pltpu.VMEM((P + 16,), jnp.int32),
            pltpu.VMEM((P + 16,), jnp.int32),
            pltpu.VMEM((P + 32,), jnp.int32),
            pltpu.VMEM((n_chunks, _CROWS), jnp.int32),
            pltpu.VMEM((_CROWS,), jnp.int32),
            pltpu.VMEM((_NBUF, _CROWS, 128), jnp.float32),
            pltpu.VMEM((_CROWS, 128), jnp.float32),
            pltpu.VMEM((P * _PATCH_FLAT,), jnp.float32),
            [pltpu.SemaphoreType.DMA] * _NBUF,
            pltpu.SemaphoreType.DMA,
        ],
        compiler_params=pltpu.CompilerParams(needs_layout_passes=False),
    )
    def gather_kernel(x_hbm, sl_hbm, sc_hbm, out_hbm, slv, scv, bids,
                      idxall, idx2, rows, rows2, outv, sems, sem2):
        wid = lax.axis_index("s") * _NC + lax.axis_index("c")
        base = wid * P
        pltpu.sync_copy(sl_hbm.at[pl.ds(base, P)], slv.at[pl.ds(0, P)])
        pltpu.sync_copy(sc_hbm.at[pl.ds(base, P)], scv.at[pl.ds(0, P)])
        rbase = wid * C
        iota = lax.iota(jnp.int32, 16)
        jv8 = lax.bitwise_and(iota, 7)
        nsel = lax.shift_right_logical(iota, 3)
        zeros = iota * 0
        ivecs = [2 * k + lax.shift_right_logical(iota, 3)
                 for k in range(_PATCH_FLAT // 16)]

        def classify(v, cnt):
            sll16 = lax.bitwise_and(slv[pl.ds(v * 16, 16)], 127)
            m = sll16 > 112
            plsc.store_compressed(bids.at[pl.ds(cnt, 16)], v * 16 + iota,
                                  mask=m)
            return cnt + plsc.all_reduce_population_count(m)[0]

        cntb = lax.fori_loop(0, P // 16, classify, jnp.int32(0))
        nb2 = lax.shift_right_logical(cntb + 15, 4)

        @pl.when(cntb > 0)
        def _():
            bids[pl.ds(cntb, 16)] = plsc.load_gather(bids, [zeros])

        def build_idx1(ci, carry):
            for v in range(_CHUNK // 2):
                nidx = ci * _CHUNK + 2 * v + nsel
                scg = plsc.load_gather(scv, [nidx])
                slg = plsc.load_gather(slv, [nidx])
                bcv = rbase + scg + jv8
                lbv = lax.shift_right_logical(slg, 7)
                rphys = (lax.shift_left(lax.shift_right_logical(bcv, 3), 9)
                         + lax.shift_left(lbv, 3) + lax.bitwise_and(bcv, 7))
                idxall[ci, pl.ds(v * 16, 16)] = rphys
            return carry

        lax.fori_loop(0, n_chunks, build_idx1, 0)

        def build_fire2(c2):
            for v in range(_CHUNK // 2):
                lsel = c2 * _CHUNK + 2 * v + nsel
                pid = plsc.load_gather(bids, [lsel])
                scg = plsc.load_gather(scv, [pid])
                slg = plsc.load_gather(slv, [pid])
                bcv = rbase + scg + jv8
                lbv = jnp.minimum(lax.shift_right_logical(slg, 7) + 1, lb_max)
                rphys = (lax.shift_left(lax.shift_right_logical(bcv, 3), 9)
                         + lax.shift_left(lbv, 3) + lax.bitwise_and(bcv, 7))
                idx2[pl.ds(v * 16, 16)] = rphys
            pltpu.async_copy(x_hbm.at[idx2], rows2, sem2)

        def extract2(c2):
            pltpu.make_async_copy(x_hbm.at[pl.ds(0, _CROWS)], rows2,
                                  sem2).wait()
            ids = plsc.load_gather(bids, [c2 * _CHUNK + iota])
            for n in range(_CHUNK):
                pid = ids[n]
                t0 = lax.bitwise_and(
                    plsc.load_gather(slv, [jnp.full((16,), pid, jnp.int32)]),
                    127)
                rvn = n * 8 + jv8
                for k in range(_PATCH_FLAT // 16):
                    t = t0 + ivecs[k]
                    m = t >= 128
                    cv = lax.bitwise_and(t, 127)
                    v = plsc.load_gather(rows2, [rvn, cv])
                    pos = pid * _PATCH_FLAT + k * 16 + iota
                    plsc.store_scatter(outv, [pos], v, mask=m)

        @pl.when(nb2 >= 1)
        def _():
            build_fire2(0)

        def fire(ci, b):
            pltpu.async_copy(x_hbm.at[idxall.at[ci]], rows.at[b], sems[b])

        def extract(ci, b):
            pltpu.make_async_copy(x_hbm.at[pl.ds(0, _CROWS)], rows.at[b],
                                  sems[b]).wait()
            slc = slv[pl.ds(ci * _CHUNK, 16)]
            for n in range(_CHUNK):
                q = ci * _CHUNK + n
                t0 = jnp.full((16,), lax.bitwise_and(slc[n], 127), jnp.int32)
                rvn = n * 8 + jv8
                for k in range(_PATCH_FLAT // 16):
                    t = t0 + ivecs[k]
                    cv = lax.bitwise_and(t, 127)
                    v = plsc.load_gather(rows.at[b], [rvn, cv])
                    outv[pl.ds(q * _PATCH_FLAT + k * 16, 16)] = v

        for b in range(_NBUF):
            fire(b, b)

        def ring_body(i, carry):
            ci = i * _NBUF
            for b in range(_NBUF):
                extract(ci + b, b)

                @pl.when(ci + b + _NBUF < n_chunks)
                def _():
                    fire(ci + b + _NBUF, b)

            return carry

        lax.fori_loop(0, n_chunks // _NBUF, ring_body, 0)

        @pl.when(nb2 >= 1)
        def _():
            extract2(0)

        def pass2_body(c2, carry):
            build_fire2(c2)
            extract2(c2)
            return carry

        lax.fori_loop(1, nb2, pass2_body, 0)

        pltpu.sync_copy(outv, out_hbm.at[pl.ds(base * _PATCH_FLAT,
                                               P * _PATCH_FLAT)])

    return gather_kernel(xp, sl, sc)


def _tc_mlp(pm, w1x, w1t, slf, inv_fs, b1, w2, b2):
    n = pm.shape[0]
    blk = 4096
    grid = (n // blk,)

    def body(inv_ref, p_ref, w1x_ref, w1t_ref, sl_ref, b1_ref, w2_ref,
             b2_ref, o_ref):
        w1t = w1t_ref[...]
        ivec = lax.shift_right_logical(
            lax.broadcasted_iota(jnp.int32, (_PATCH_FLAT, 1), 0), 3
        ).astype(jnp.float32)
        s0 = jnp.sum(w1t, axis=0, keepdims=True)
        s1 = jnp.sum(w1t * ivec, axis=0, keepdims=True)
        inv = inv_ref[0, 0]
        h = jnp.dot(p_ref[...], w1x_ref[...],
                    preferred_element_type=jnp.float32)
        h = h + (sl_ref[...] * inv) * s0 + (inv * s1 + b1_ref[...])
        h = h * jax.nn.sigmoid(h)
        o_ref[...] = jnp.dot(h, w2_ref[...],
                             preferred_element_type=jnp.float32) + b2_ref[...]

    return pl.pallas_call(
        body,
        grid=grid,
        in_specs=[
            pl.BlockSpec(memory_space=pltpu.SMEM),
            pl.BlockSpec((blk, _PATCH_FLAT), lambda i: (i, 0)),
            pl.BlockSpec((_PATCH_FLAT, _OUT_DIM), lambda i: (0, 0)),
            pl.BlockSpec((_PATCH_FLAT, _OUT_DIM), lambda i: (0, 0)),
            pl.BlockSpec((blk, 1), lambda i: (i, 0)),
            pl.BlockSpec((1, _OUT_DIM), lambda i: (0, 0)),
            pl.BlockSpec((_OUT_DIM, _OUT_DIM), lambda i: (0, 0)),
            pl.BlockSpec((1, _OUT_DIM), lambda i: (0, 0)),
        ],
        out_specs=pl.BlockSpec((blk, _OUT_DIM), lambda i: (i, 0)),
        out_shape=jax.ShapeDtypeStruct((n, _OUT_DIM), jnp.float32),
    )(inv_fs, pm, w1x, w1t, slf, b1, w2, b2)


@functools.lru_cache(maxsize=None)
def _patch_starts(B, L, C):
    try:
        with jax.ensure_compile_time_eval():
            kidx = jax.random.key(42)
            kL, kC = jax.random.split(kidx)
            start_l = jax.random.randint(kL, (B, _NUM_PATCHES), 0,
                                         L - _PATCH_L + 1)
            start_c = jax.random.randint(kC, (B, _NUM_PATCHES), 0,
                                         C - _PATCH_C + 1)
            sl = np.asarray(start_l, np.int32).reshape(-1)
            sc = np.asarray(start_c, np.int32).reshape(-1)
        return sl, sc
    except Exception:
        return None


def _patch_starts_traced(B, L, C):
    kidx = jax.random.key(42)
    kL, kC = jax.random.split(kidx)
    start_l = jax.random.randint(kL, (B, _NUM_PATCHES), 0, L - _PATCH_L + 1)
    start_c = jax.random.randint(kC, (B, _NUM_PATCHES), 0, C - _PATCH_C + 1)
    return (start_l.reshape(-1).astype(jnp.int32),
            start_c.reshape(-1).astype(jnp.int32))


def kernel(x, fs, W1, b1, W2, b2):
    B, L, C = x.shape
    starts = _patch_starts(B, L, C)
    if starts is None:
        sl, sc = _patch_starts_traced(B, L, C)
        slf = sl.astype(jnp.float32).reshape(-1, 1)
    else:
        sl_np, sc_np = starts
        sl = jnp.asarray(sl_np)
        sc = jnp.asarray(sc_np)
        slf = jnp.asarray(sl_np.astype(np.float32).reshape(-1, 1))

    xp = (x.reshape(B, L // 128, 128, C // 8, 8).transpose(0, 3, 1, 4, 2)
          .reshape(B * L * C // 128, 128))
    pm = _sc_gather_patches(xp, sl, sc, L, C)
    pm = pm.reshape(B * _NUM_PATCHES, _PATCH_FLAT)

    w1r = W1.reshape(_PATCH_L, 2 * _PATCH_C, _OUT_DIM)
    w1x = w1r[:, :_PATCH_C, :].reshape(_PATCH_FLAT, _OUT_DIM)
    w1t = w1r[:, _PATCH_C:, :].reshape(_PATCH_FLAT, _OUT_DIM)
    inv_fs = (1.0 / jnp.asarray(fs).astype(jnp.float32)).reshape(1, 1)

    out = _tc_mlp(pm, w1x, w1t, slf, inv_fs, b1.reshape(1, -1), W2,
                  b2.reshape(1, -1))
    return out.reshape(B, _NUM_PATCHES, _OUT_DIM)


# speedup vs baseline: 1.1233x; 1.1233x over previous
import functools

import jax
import jax.numpy as jnp
import numpy as np
from jax import lax
from jax.experimental import pallas as pl
from jax.experimental.pallas import tpu as pltpu
from jax.experimental.pallas import tpu_sc as plsc

_PATCH_L = 16
_PATCH_C = 8
_NUM_PATCHES = 256
_OUT_DIM = 128
_PATCH_FLAT = _PATCH_L * _PATCH_C

_NC, _NS = 2, 16
_NW = _NC * _NS
_CHUNK = 16
_CROWS = _CHUNK * 8
_NBUF = 4


def _sc_gather_patches(xp, sl, sc, L, C):
    W = xp.shape[0]
    NP = sl.shape[0]
    P = NP // _NW
    n_chunks = P // _CHUNK
    lb_max = L // 128 - 1

    mesh = plsc.VectorSubcoreMesh(core_axis_name="c", subcore_axis_name="s")

    @functools.partial(
        pl.kernel,
        out_type=jax.ShapeDtypeStruct((NP * _PATCH_FLAT,), jnp.float32),
        mesh=mesh,
        scratch_types=[
            pltpu.VMEM((P + 16,), jnp.int32),
            pltpu.VMEM((P + 16,), jnp.int32),
            pltpu.VMEM((P + 32,), jnp.int32),
            pltpu.VMEM((n_chunks, _CROWS), jnp.int32),
            pltpu.VMEM((_CROWS,), jnp.int32),
            pltpu.VMEM((_NBUF, _CROWS, 128), jnp.float32),
            pltpu.VMEM((_CROWS, 128), jnp.float32),
            pltpu.VMEM((P * _PATCH_FLAT,), jnp.float32),
            [pltpu.SemaphoreType.DMA] * _NBUF,
            pltpu.SemaphoreType.DMA,
        ],
        compiler_params=pltpu.CompilerParams(needs_layout_passes=False),
    )
    def gather_kernel(x_hbm, sl_hbm, sc_hbm, out_hbm, slv, scv, bids,
                      idxall, idx2, rows, rows2, outv, sems, sem2):
        wid = lax.axis_index("s") * _NC + lax.axis_index("c")
        base = wid * P
        pltpu.sync_copy(sl_hbm.at[pl.ds(base, P)], slv.at[pl.ds(0, P)])
        pltpu.sync_copy(sc_hbm.at[pl.ds(base, P)], scv.at[pl.ds(0, P)])
        rbase = wid * C
        iota = lax.iota(jnp.int32, 16)
        jv8 = lax.bitwise_and(iota, 7)
        nsel = lax.shift_right_logical(iota, 3)
        zeros = iota * 0
        ivecs = [2 * k + lax.shift_right_logical(iota, 3)
                 for k in range(_PATCH_FLAT // 16)]

        def classify(v, cnt):
            sll16 = lax.bitwise_and(slv[pl.ds(v * 16, 16)], 127)
            m = sll16 > 112
            plsc.store_compressed(bids.at[pl.ds(cnt, 16)], v * 16 + iota,
                                  mask=m)
            return cnt + plsc.all_reduce_population_count(m)[0]

        cntb = lax.fori_loop(0, P // 16, classify, jnp.int32(0))
        nb2 = lax.shift_right_logical(cntb + 15, 4)

        @pl.when(cntb > 0)
        def _():
            bids[pl.ds(cntb, 16)] = plsc.load_gather(bids, [zeros])

        def build_idx1(ci, carry):
            for v in range(_CHUNK // 2):
                nidx = ci * _CHUNK + 2 * v + nsel
                scg = plsc.load_gather(scv, [nidx])
                slg = plsc.load_gather(slv, [nidx])
                bcv = rbase + scg + jv8
                lbv = lax.shift_right_logical(slg, 7)
                rphys = (lax.shift_left(lax.shift_right_logical(bcv, 3), 9)
                         + lax.shift_left(lbv, 3) + lax.bitwise_and(bcv, 7))
                idxall[ci, pl.ds(v * 16, 16)] = rphys
            return carry

        lax.fori_loop(0, n_chunks, build_idx1, 0)

        def build_fire2(c2):
            for v in range(_CHUNK // 2):
                lsel = c2 * _CHUNK + 2 * v + nsel
                pid = plsc.load_gather(bids, [lsel])
                scg = plsc.load_gather(scv, [pid])
                slg = plsc.load_gather(slv, [pid])
                bcv = rbase + scg + jv8
                lbv = jnp.minimum(lax.shift_right_logical(slg, 7) + 1, lb_max)
                rphys = (lax.shift_left(lax.shift_right_logical(bcv, 3), 9)
                         + lax.shift_left(lbv, 3) + lax.bitwise_and(bcv, 7))
                idx2[pl.ds(v * 16, 16)] = rphys
            pltpu.async_copy(x_hbm.at[idx2], rows2, sem2)

        def extract2(c2):
            pltpu.make_async_copy(x_hbm.at[pl.ds(0, _CROWS)], rows2,
                                  sem2).wait()
            ids = plsc.load_gather(bids, [c2 * _CHUNK + iota])
            for n in range(_CHUNK):
                pid = ids[n]
                t0 = lax.bitwise_and(
                    plsc.load_gather(slv, [jnp.full((16,), pid, jnp.int32)]),
                    127)
                rvn = n * 8 + jv8
                for k in range(_PATCH_FLAT // 16):
                    t = t0 + ivecs[k]
                    m = t >= 128
                    cv = lax.bitwise_and(t, 127)
                    v = plsc.load_gather(rows2, [rvn, cv])
                    pos = pid * _PATCH_FLAT + k * 16 + iota
                    plsc.store_scatter(outv, [pos], v, mask=m)

        @pl.when(nb2 >= 1)
        def _():
            build_fire2(0)

        def fire(ci, b):
            pltpu.async_copy(x_hbm.at[idxall.at[ci]], rows.at[b], sems[b])

        def extract(ci, b):
            pltpu.make_async_copy(x_hbm.at[pl.ds(0, _CROWS)], rows.at[b],
                                  sems[b]).wait()
            slc = slv[pl.ds(ci * _CHUNK, 16)]
            for n2 in range(_CHUNK // 2):
                vs = []
                for n in (2 * n2, 2 * n2 + 1):
                    t0 = jnp.full((16,), lax.bitwise_and(slc[n], 127),
                                  jnp.int32)
                    rvn = n * 8 + jv8
                    for k in range(_PATCH_FLAT // 16):
                        t = t0 + ivecs[k]
                        cv = lax.bitwise_and(t, 127)
                        vs.append(plsc.load_gather(rows.at[b], [rvn, cv]))
                for n in (2 * n2, 2 * n2 + 1):
                    q = ci * _CHUNK + n
                    for k in range(_PATCH_FLAT // 16):
                        outv[pl.ds(q * _PATCH_FLAT + k * 16, 16)] = (
                            vs[(n & 1) * 8 + k])

        for b in range(_NBUF):
            fire(b, b)

        def ring_body(i, carry):
            ci = i * _NBUF
            for b in range(_NBUF):
                extract(ci + b, b)

                @pl.when(ci + b + _NBUF < n_chunks)
                def _():
                    fire(ci + b + _NBUF, b)

            return carry

        lax.fori_loop(0, n_chunks // _NBUF, ring_body, 0)

        @pl.when(nb2 >= 1)
        def _():
            extract2(0)

        def pass2_body(c2, carry):
            build_fire2(c2)
            extract2(c2)
            return carry

        lax.fori_loop(1, nb2, pass2_body, 0)

        pltpu.sync_copy(outv, out_hbm.at[pl.ds(base * _PATCH_FLAT,
                                               P * _PATCH_FLAT)])

    return gather_kernel(xp, sl, sc)


def _tc_mlp(pm, w1x, w1t, slf, inv_fs, b1, w2, b2):
    n = pm.shape[0]
    blk = 4096
    grid = (n // blk,)

    def body(inv_ref, p_ref, w1x_ref, w1t_ref, sl_ref, b1_ref, w2_ref,
             b2_ref, o_ref):
        w1t = w1t_ref[...]
        ivec = lax.shift_right_logical(
            lax.broadcasted_iota(jnp.int32, (_PATCH_FLAT, 1), 0), 3
        ).astype(jnp.float32)
        s0 = jnp.sum(w1t, axis=0, keepdims=True)
        s1 = jnp.sum(w1t * ivec, axis=0, keepdims=True)
        inv = inv_ref[0, 0]
        h = jnp.dot(p_ref[...], w1x_ref[...],
                    preferred_element_type=jnp.float32)
        h = h + (sl_ref[...] * inv) * s0 + (inv * s1 + b1_ref[...])
        h = h * jax.nn.sigmoid(h)
        o_ref[...] = jnp.dot(h, w2_ref[...],
                             preferred_element_type=jnp.float32) + b2_ref[...]

    return pl.pallas_call(
        body,
        grid=grid,
        in_specs=[
            pl.BlockSpec(memory_space=pltpu.SMEM),
            pl.BlockSpec((blk, _PATCH_FLAT), lambda i: (i, 0)),
            pl.BlockSpec((_PATCH_FLAT, _OUT_DIM), lambda i: (0, 0)),
            pl.BlockSpec((_PATCH_FLAT, _OUT_DIM), lambda i: (0, 0)),
            pl.BlockSpec((blk, 1), lambda i: (i, 0)),
            pl.BlockSpec((1, _OUT_DIM), lambda i: (0, 0)),
            pl.BlockSpec((_OUT_DIM, _OUT_DIM), lambda i: (0, 0)),
            pl.BlockSpec((1, _OUT_DIM), lambda i: (0, 0)),
        ],
        out_specs=pl.BlockSpec((blk, _OUT_DIM), lambda i: (i, 0)),
        out_shape=jax.ShapeDtypeStruct((n, _OUT_DIM), jnp.float32),
    )(inv_fs, pm, w1x, w1t, slf, b1, w2, b2)


@functools.lru_cache(maxsize=None)
def _patch_starts(B, L, C):
    try:
        with jax.ensure_compile_time_eval():
            kidx = jax.random.key(42)
            kL, kC = jax.random.split(kidx)
            start_l = jax.random.randint(kL, (B, _NUM_PATCHES), 0,
                                         L - _PATCH_L + 1)
            start_c = jax.random.randint(kC, (B, _NUM_PATCHES), 0,
                                         C - _PATCH_C + 1)
            sl = np.asarray(start_l, np.int32).reshape(-1)
            sc = np.asarray(start_c, np.int32).reshape(-1)
        return sl, sc
    except Exception:
        return None


def _patch_starts_traced(B, L, C):
    kidx = jax.random.key(42)
    kL, kC = jax.random.split(kidx)
    start_l = jax.random.randint(kL, (B, _NUM_PATCHES), 0, L - _PATCH_L + 1)
    start_c = jax.random.randint(kC, (B, _NUM_PATCHES), 0, C - _PATCH_C + 1)
    return (start_l.reshape(-1).astype(jnp.int32),
            start_c.reshape(-1).astype(jnp.int32))


def kernel(x, fs, W1, b1, W2, b2):
    B, L, C = x.shape
    starts = _patch_starts(B, L, C)
    if starts is None:
        sl, sc = _patch_starts_traced(B, L, C)
        slf = sl.astype(jnp.float32).reshape(-1, 1)
    else:
        sl_np, sc_np = starts
        sl = jnp.asarray(sl_np)
        sc = jnp.asarray(sc_np)
        slf = jnp.asarray(sl_np.astype(np.float32).reshape(-1, 1))

    xp = (x.reshape(B, L // 128, 128, C // 8, 8).transpose(0, 3, 1, 4, 2)
          .reshape(B * L * C // 128, 128))
    pm = _sc_gather_patches(xp, sl, sc, L, C)
    pm = pm.reshape(B * _NUM_PATCHES, _PATCH_FLAT)

    w1r = W1.reshape(_PATCH_L, 2 * _PATCH_C, _OUT_DIM)
    w1x = w1r[:, :_PATCH_C, :].reshape(_PATCH_FLAT, _OUT_DIM)
    w1t = w1r[:, _PATCH_C:, :].reshape(_PATCH_FLAT, _OUT_DIM)
    inv_fs = (1.0 / jnp.asarray(fs).astype(jnp.float32)).reshape(1, 1)

    out = _tc_mlp(pm, w1x, w1t, slf, inv_fs, b1.reshape(1, -1), W2,
                  b2.reshape(1, -1))
    return out.reshape(B, _NUM_PATCHES, _OUT_DIM)


# speedup vs baseline: 1.1270x; 1.0033x over previous
import functools

import jax
import jax.numpy as jnp
import numpy as np
from jax import lax
from jax.experimental import pallas as pl
from jax.experimental.pallas import tpu as pltpu
from jax.experimental.pallas import tpu_sc as plsc

_PATCH_L = 16
_PATCH_C = 8
_NUM_PATCHES = 256
_OUT_DIM = 128
_PATCH_FLAT = _PATCH_L * _PATCH_C

_NC, _NS = 2, 16
_NW = _NC * _NS
_CHUNK = 16
_CROWS = _CHUNK * 8
_NBUF = 4


def _sc_gather_patches(xp, sl, sc, L, C):
    W = xp.shape[0]
    NP = sl.shape[0]
    P = NP // _NW
    n_chunks = P // _CHUNK
    lb_max = L // 128 - 1

    mesh = plsc.VectorSubcoreMesh(core_axis_name="c", subcore_axis_name="s")

    @functools.partial(
        pl.kernel,
        out_type=jax.ShapeDtypeStruct((NP * _PATCH_FLAT,), jnp.float32),
        mesh=mesh,
        scratch_types=[
            pltpu.VMEM((P + 16,), jnp.int32),
            pltpu.VMEM((P + 16,), jnp.int32),
            pltpu.VMEM((P + 32,), jnp.int32),
            pltpu.VMEM((n_chunks, _CROWS), jnp.int32),
            pltpu.VMEM((_CROWS,), jnp.int32),
            pltpu.VMEM((_NBUF, _CROWS, 128), jnp.float32),
            pltpu.VMEM((_CROWS, 128), jnp.float32),
            pltpu.VMEM((P * _PATCH_FLAT,), jnp.float32),
            [pltpu.SemaphoreType.DMA] * _NBUF,
            pltpu.SemaphoreType.DMA,
        ],
        compiler_params=pltpu.CompilerParams(needs_layout_passes=False),
    )
    def gather_kernel(x_hbm, sl_hbm, sc_hbm, out_hbm, slv, scv, bids,
                      idxall, idx2, rows, rows2, outv, sems, sem2):
        wid = lax.axis_index("s") * _NC + lax.axis_index("c")
        base = wid * P
        pltpu.sync_copy(sl_hbm.at[pl.ds(base, P)], slv.at[pl.ds(0, P)])
        pltpu.sync_copy(sc_hbm.at[pl.ds(base, P)], scv.at[pl.ds(0, P)])
        rbase = wid * C
        iota = lax.iota(jnp.int32, 16)
        jv8 = lax.bitwise_and(iota, 7)
        nsel = lax.shift_right_logical(iota, 3)
        zeros = iota * 0
        ivecs = [2 * k + lax.shift_right_logical(iota, 3)
                 for k in range(_PATCH_FLAT // 16)]

        def classify(v, cnt):
            sll16 = lax.bitwise_and(slv[pl.ds(v * 16, 16)], 127)
            m = sll16 > 112
            plsc.store_compressed(bids.at[pl.ds(cnt, 16)], v * 16 + iota,
                                  mask=m)
            return cnt + plsc.all_reduce_population_count(m)[0]

        cntb = lax.fori_loop(0, P // 16, classify, jnp.int32(0))
        nb2 = lax.shift_right_logical(cntb + 15, 4)

        @pl.when(cntb > 0)
        def _():
            bids[pl.ds(cntb, 16)] = plsc.load_gather(bids, [zeros])

        def build_idx1(ci, carry):
            for v in range(_CHUNK // 2):
                nidx = ci * _CHUNK + 2 * v + nsel
                scg = plsc.load_gather(scv, [nidx])
                slg = plsc.load_gather(slv, [nidx])
                bcv = rbase + scg + jv8
                lbv = lax.shift_right_logical(slg, 7)
                rphys = (lax.shift_left(lax.shift_right_logical(bcv, 3), 9)
                         + lax.shift_left(lbv, 3) + lax.bitwise_and(bcv, 7))
                idxall[ci, pl.ds(v * 16, 16)] = rphys
            return carry

        lax.fori_loop(0, n_chunks, build_idx1, 0)

        def build_fire2(c2):
            for v in range(_CHUNK // 2):
                lsel = c2 * _CHUNK + 2 * v + nsel
                pid = plsc.load_gather(bids, [lsel])
                scg = plsc.load_gather(scv, [pid])
                slg = plsc.load_gather(slv, [pid])
                bcv = rbase + scg + jv8
                lbv = jnp.minimum(lax.shift_right_logical(slg, 7) + 1, lb_max)
                rphys = (lax.shift_left(lax.shift_right_logical(bcv, 3), 9)
                         + lax.shift_left(lbv, 3) + lax.bitwise_and(bcv, 7))
                idx2[pl.ds(v * 16, 16)] = rphys
            pltpu.async_copy(x_hbm.at[idx2], rows2, sem2)

        def extract2(c2):
            pltpu.make_async_copy(x_hbm.at[pl.ds(0, _CROWS)], rows2,
                                  sem2).wait()
            ids = plsc.load_gather(bids, [c2 * _CHUNK + iota])
            for n in range(_CHUNK):
                pid = ids[n]
                t0 = lax.bitwise_and(
                    plsc.load_gather(slv, [jnp.full((16,), pid, jnp.int32)]),
                    127)
                rvn = n * 8 + jv8
                for k in range(_PATCH_FLAT // 16):
                    t = t0 + ivecs[k]
                    m = t >= 128
                    cv = lax.bitwise_and(t, 127)
                    v = plsc.load_gather(rows2, [rvn, cv])
                    pos = pid * _PATCH_FLAT + k * 16 + iota
                    plsc.store_scatter(outv, [pos], v, mask=m)

        @pl.when(nb2 >= 1)
        def _():
            build_fire2(0)

        def fire(ci, b):
            pltpu.async_copy(x_hbm.at[idxall.at[ci]], rows.at[b], sems[b])

        def extract(ci, b):
            pltpu.make_async_copy(x_hbm.at[pl.ds(0, _CROWS)], rows.at[b],
                                  sems[b]).wait()
            slc = slv[pl.ds(ci * _CHUNK, 16)]
            nb = 4
            for n2 in range(_CHUNK // nb):
                vs = []
                for dn in range(nb):
                    n = nb * n2 + dn
                    t0 = jnp.full((16,), lax.bitwise_and(slc[n], 127),
                                  jnp.int32)
                    rvn = n * 8 + jv8
                    for k in range(_PATCH_FLAT // 16):
                        t = t0 + ivecs[k]
                        cv = lax.bitwise_and(t, 127)
                        vs.append(plsc.load_gather(rows.at[b], [rvn, cv]))
                for dn in range(nb):
                    q = ci * _CHUNK + nb * n2 + dn
                    for k in range(_PATCH_FLAT // 16):
                        outv[pl.ds(q * _PATCH_FLAT + k * 16, 16)] = (
                            vs[dn * 8 + k])

        for b in range(_NBUF):
            fire(b, b)

        def ring_body(i, carry):
            ci = i * _NBUF
            for b in range(_NBUF):
                extract(ci + b, b)

                @pl.when(ci + b + _NBUF < n_chunks)
                def _():
                    fire(ci + b + _NBUF, b)

            return carry

        lax.fori_loop(0, n_chunks // _NBUF, ring_body, 0)

        @pl.when(nb2 >= 1)
        def _():
            extract2(0)

        def pass2_body(c2, carry):
            build_fire2(c2)
            extract2(c2)
            return carry

        lax.fori_loop(1, nb2, pass2_body, 0)

        pltpu.sync_copy(outv, out_hbm.at[pl.ds(base * _PATCH_FLAT,
                                               P * _PATCH_FLAT)])

    return gather_kernel(xp, sl, sc)


def _tc_mlp(pm, w1x, w1t, slf, inv_fs, b1, w2, b2):
    n = pm.shape[0]
    blk = 4096
    grid = (n // blk,)

    def body(inv_ref, p_ref, w1x_ref, w1t_ref, sl_ref, b1_ref, w2_ref,
             b2_ref, o_ref):
        w1t = w1t_ref[...]
        ivec = lax.shift_right_logical(
            lax.broadcasted_iota(jnp.int32, (_PATCH_FLAT, 1), 0), 3
        ).astype(jnp.float32)
        s0 = jnp.sum(w1t, axis=0, keepdims=True)
        s1 = jnp.sum(w1t * ivec, axis=0, keepdims=True)
        inv = inv_ref[0, 0]
        h = jnp.dot(p_ref[...], w1x_ref[...],
                    preferred_element_type=jnp.float32)
        h = h + (sl_ref[...] * inv) * s0 + (inv * s1 + b1_ref[...])
        h = h * jax.nn.sigmoid(h)
        o_ref[...] = jnp.dot(h, w2_ref[...],
                             preferred_element_type=jnp.float32) + b2_ref[...]

    return pl.pallas_call(
        body,
        grid=grid,
        in_specs=[
            pl.BlockSpec(memory_space=pltpu.SMEM),
            pl.BlockSpec((blk, _PATCH_FLAT), lambda i: (i, 0)),
            pl.BlockSpec((_PATCH_FLAT, _OUT_DIM), lambda i: (0, 0)),
            pl.BlockSpec((_PATCH_FLAT, _OUT_DIM), lambda i: (0, 0)),
            pl.BlockSpec((blk, 1), lambda i: (i, 0)),
            pl.BlockSpec((1, _OUT_DIM), lambda i: (0, 0)),
            pl.BlockSpec((_OUT_DIM, _OUT_DIM), lambda i: (0, 0)),
            pl.BlockSpec((1, _OUT_DIM), lambda i: (0, 0)),
        ],
        out_specs=pl.BlockSpec((blk, _OUT_DIM), lambda i: (i, 0)),
        out_shape=jax.ShapeDtypeStruct((n, _OUT_DIM), jnp.float32),
    )(inv_fs, pm, w1x, w1t, slf, b1, w2, b2)


@functools.lru_cache(maxsize=None)
def _patch_starts(B, L, C):
    try:
        with jax.ensure_compile_time_eval():
            kidx = jax.random.key(42)
            kL, kC = jax.random.split(kidx)
            start_l = jax.random.randint(kL, (B, _NUM_PATCHES), 0,
                                         L - _PATCH_L + 1)
            start_c = jax.random.randint(kC, (B, _NUM_PATCHES), 0,
                                         C - _PATCH_C + 1)
            sl = np.asarray(start_l, np.int32).reshape(-1)
            sc = np.asarray(start_c, np.int32).reshape(-1)
        return sl, sc
    except Exception:
        return None


def _patch_starts_traced(B, L, C):
    kidx = jax.random.key(42)
    kL, kC = jax.random.split(kidx)
    start_l = jax.random.randint(kL, (B, _NUM_PATCHES), 0, L - _PATCH_L + 1)
    start_c = jax.random.randint(kC, (B, _NUM_PATCHES), 0, C - _PATCH_C + 1)
    return (start_l.reshape(-1).astype(jnp.int32),
            start_c.reshape(-1).astype(jnp.int32))


def kernel(x, fs, W1, b1, W2, b2):
    B, L, C = x.shape
    starts = _patch_starts(B, L, C)
    if starts is None:
        sl, sc = _patch_starts_traced(B, L, C)
        slf = sl.astype(jnp.float32).reshape(-1, 1)
    else:
        sl_np, sc_np = starts
        sl = jnp.asarray(sl_np)
        sc = jnp.asarray(sc_np)
        slf = jnp.asarray(sl_np.astype(np.float32).reshape(-1, 1))

    xp = (x.reshape(B, L // 128, 128, C // 8, 8).transpose(0, 3, 1, 4, 2)
          .reshape(B * L * C // 128, 128))
    pm = _sc_gather_patches(xp, sl, sc, L, C)
    pm = pm.reshape(B * _NUM_PATCHES, _PATCH_FLAT)

    w1r = W1.reshape(_PATCH_L, 2 * _PATCH_C, _OUT_DIM)
    w1x = w1r[:, :_PATCH_C, :].reshape(_PATCH_FLAT, _OUT_DIM)
    w1t = w1r[:, _PATCH_C:, :].reshape(_PATCH_FLAT, _OUT_DIM)
    inv_fs = (1.0 / jnp.asarray(fs).astype(jnp.float32)).reshape(1, 1)

    out = _tc_mlp(pm, w1x, w1t, slf, inv_fs, b1.reshape(1, -1), W2,
                  b2.reshape(1, -1))
    return out.reshape(B, _NUM_PATCHES, _OUT_DIM)
